# initial kernel scaffold (unmeasured)
import jax
import jax.numpy as jnp
from jax import lax
from jax.experimental import pallas as pl
from jax.experimental.pallas import tpu as pltpu

N_DEV = 32
N_TOK = 1024
D_IN = 512
D_OUT = 1024
E_PER_DEV = 4
CAP = 6
SLOTS_PER_E = 8
ROWS = E_PER_DEV * SLOTS_PER_E


def kernel(x, router_W, route_idx, expert_W):
    del router_W
    x_bf = x.astype(jnp.bfloat16)
    w_bf = expert_W.astype(jnp.bfloat16)

    def body(x_ref, idx_ref, w_ref, out_ref, comm_ref, send_sems, recv_sems):
        my = lax.axis_index("i")

        e_tok = idx_ref[:, :]
        ti = lax.broadcasted_iota(jnp.int32, (N_TOK, N_TOK), 0)
        tj = lax.broadcasted_iota(jnp.int32, (N_TOK, N_TOK), 1)
        same = (e_tok == e_tok.reshape(1, N_TOK)) & (tj < ti)
        cb = jnp.sum(same.astype(jnp.float32), axis=1, keepdims=True)

        g = lax.broadcasted_iota(jnp.int32, (N_DEV * ROWS, N_TOK), 0)
        origin = jnp.bitwise_xor(g // ROWS, my)
        e_row = E_PER_DEV * origin + (g % ROWS) // SLOTS_PER_E
        k_row = g % SLOTS_PER_E
        e_col = e_tok.reshape(1, N_TOK)
        cb_col = cb.reshape(1, N_TOK)
        S = (
            (e_col == e_row)
            & (cb_col == k_row.astype(jnp.float32))
            & (cb_col < CAP)
        ).astype(jnp.bfloat16)

        s_local = S[0:ROWS, :]
        x_c = jnp.dot(
            s_local, x_ref[:, :], preferred_element_type=jnp.float32
        ).astype(jnp.bfloat16)
        c_rows = []
        for le in range(E_PER_DEV):
            c_rows.append(
                jnp.dot(
                    x_c[le * SLOTS_PER_E : (le + 1) * SLOTS_PER_E, :],
                    w_ref[le],
                    preferred_element_type=jnp.float32,
                )
            )
        c_local = jnp.concatenate(c_rows, axis=0).astype(jnp.bfloat16)
        comm_ref[0:ROWS, :] = c_local

        for k in range(5):
            block = 1 << k
            partner = jnp.bitwise_xor(my, block)
            rdma = pltpu.make_async_remote_copy(
                src_ref=comm_ref.at[pl.ds(0, ROWS * block), :],
                dst_ref=comm_ref.at[pl.ds(ROWS * block, ROWS * block), :],
                send_sem=send_sems.at[k],
                recv_sem=recv_sems.at[k],
                device_id=(partner,),
                device_id_type=pl.DeviceIdType.MESH,
            )
            rdma.start()
            rdma.wait()

        out_ref[:, :] = lax.dot_general(
            S,
            comm_ref[:, :],
            dimension_numbers=(((0,), (0,)), ((), ())),
            preferred_element_type=jnp.float32,
        )

    return pl.pallas_call(
        body,
        out_shape=jax.ShapeDtypeStruct((N_TOK, D_OUT), jnp.float32),
        in_specs=[
            pl.BlockSpec(memory_space=pltpu.VMEM),
            pl.BlockSpec(memory_space=pltpu.VMEM),
            pl.BlockSpec(memory_space=pltpu.VMEM),
        ],
        out_specs=pl.BlockSpec(memory_space=pltpu.VMEM),
        scratch_shapes=[
            pltpu.VMEM((N_DEV * ROWS, D_OUT), jnp.bfloat16),
            pltpu.SemaphoreType.DMA((5,)),
            pltpu.SemaphoreType.DMA((5,)),
        ],
        compiler_params=pltpu.CompilerParams(collective_id=0),
    )(x_bf, route_idx, w_bf)


# baseline (device time: 65451 ns/iter reference)
import jax
import jax.numpy as jnp
from jax import lax
from jax.experimental import pallas as pl
from jax.experimental.pallas import tpu as pltpu

N_DEV = 32
N_TOK = 1024
D_IN = 512
D_OUT = 1024
E_PER_DEV = 4
CAP = 6
SLOTS_PER_E = 8
ROWS = E_PER_DEV * SLOTS_PER_E


def kernel(x, router_W, route_idx, expert_W):
    del router_W
    x_bf = x.astype(jnp.bfloat16)
    w_bf = expert_W.astype(jnp.bfloat16)

    def body(x_ref, idx_ref, w_ref, out_ref, comm_ref, send_sems, recv_sems):
        my = lax.axis_index("i")

        barrier_sem = pltpu.get_barrier_semaphore()
        for k in range(5):
            pl.semaphore_signal(
                barrier_sem,
                inc=1,
                device_id=(jnp.bitwise_xor(my, 1 << k),),
                device_id_type=pl.DeviceIdType.MESH,
            )
        pl.semaphore_wait(barrier_sem, 5)

        e_tok = idx_ref[:, :]
        oh = (e_tok == lax.broadcasted_iota(jnp.int32, (1, 128), 1)).astype(
            jnp.bfloat16
        )
        tri = (
            lax.broadcasted_iota(jnp.int32, (N_TOK, N_TOK), 1)
            < lax.broadcasted_iota(jnp.int32, (N_TOK, N_TOK), 0)
        ).astype(jnp.bfloat16)
        cb_full = jnp.dot(tri, oh, preferred_element_type=jnp.float32)
        cb = jnp.sum(
            oh.astype(jnp.float32) * cb_full, axis=1, keepdims=True
        ).astype(jnp.int32)

        d_tok = e_tok // E_PER_DEV
        slot = (
            ROWS * jnp.bitwise_xor(d_tok, my)
            + SLOTS_PER_E * (e_tok % E_PER_DEV)
            + cb
        )
        slot = jnp.where(cb < CAP, slot, -1)
        g = lax.broadcasted_iota(jnp.int32, (N_DEV * ROWS, 1), 0)
        S = (g == slot.reshape(1, N_TOK)).astype(jnp.bfloat16)

        s_local = S[0:ROWS, :]
        x_c = jnp.dot(
            s_local, x_ref[:, :], preferred_element_type=jnp.float32
        ).astype(jnp.bfloat16)
        c_rows = []
        for le in range(E_PER_DEV):
            c_rows.append(
                jnp.dot(
                    x_c[le * SLOTS_PER_E : (le + 1) * SLOTS_PER_E, :],
                    w_ref[le],
                    preferred_element_type=jnp.float32,
                )
            )
        c_local = jnp.concatenate(c_rows, axis=0).astype(jnp.bfloat16)
        comm_ref[0:ROWS, :] = c_local

        for k in range(5):
            block = 1 << k
            partner = jnp.bitwise_xor(my, block)
            rdma = pltpu.make_async_remote_copy(
                src_ref=comm_ref.at[pl.ds(0, ROWS * block), :],
                dst_ref=comm_ref.at[pl.ds(ROWS * block, ROWS * block), :],
                send_sem=send_sems.at[k],
                recv_sem=recv_sems.at[k],
                device_id=(partner,),
                device_id_type=pl.DeviceIdType.MESH,
            )
            rdma.start()
            rdma.wait()

        out_ref[:, :] = lax.dot_general(
            S,
            comm_ref[:, :],
            dimension_numbers=(((0,), (0,)), ((), ())),
            preferred_element_type=jnp.float32,
        )

    return pl.pallas_call(
        body,
        out_shape=jax.ShapeDtypeStruct((N_TOK, D_OUT), jnp.float32),
        in_specs=[
            pl.BlockSpec(memory_space=pltpu.VMEM),
            pl.BlockSpec(memory_space=pltpu.VMEM),
            pl.BlockSpec(memory_space=pltpu.VMEM),
        ],
        out_specs=pl.BlockSpec(memory_space=pltpu.VMEM),
        scratch_shapes=[
            pltpu.VMEM((N_DEV * ROWS, D_OUT), jnp.bfloat16),
            pltpu.SemaphoreType.DMA((5,)),
            pltpu.SemaphoreType.DMA((5,)),
        ],
        compiler_params=pltpu.CompilerParams(collective_id=0),
    )(x_bf, route_idx, w_bf)
